# baseline (speedup 1.0000x reference)
import functools

import jax
import jax.numpy as jnp
from jax import lax
from jax.experimental import pallas as pl
from jax.experimental.pallas import tpu as pltpu
from jax.experimental.pallas import tpu_sc as plsc

_H = 64
_EMB = 64
_G = 8
_NEIGH = 4.0
_NPED = 512
_NSEQ = 8
_GSN = _NEIGH / (_G - 1)

_NC = 2
_NS = 16
_NW = _NC * _NS
_IPW = _NPED // _NW
_NJV = _NPED // 16
_PACC = 80 * _H


def _sc_pool(hT_hbm, posT_hbm, seq_hbm, pooled_hbm,
             hT_v, px_v, py_v, mj_v, pacc_v, seq_v):
    G = _G
    half = G // 2
    wid = lax.axis_index("s") * _NC + lax.axis_index("c")

    pltpu.sync_copy(hT_hbm, hT_v)
    pltpu.sync_copy(posT_hbm.at[0], px_v)
    pltpu.sync_copy(posT_hbm.at[1], py_v)
    pltpu.sync_copy(seq_hbm, seq_v)

    lanes = lax.broadcasted_iota(jnp.int32, (16,), 0)
    seqvec = seq_v[pl.ds(0, 16)]

    def _lane_i32(vec, lane):
        return lax.reduce_sum_p.bind(
            jnp.where(lanes == lane, vec, 0), axes=(0,))

    sts = [_lane_i32(seqvec, 2 * s) for s in range(_NSEQ)]
    ens = [_lane_i32(seqvec, 2 * s + 1) for s in range(_NSEQ)]

    for s in range(_NSEQ):
        for k in range(_NJV):
            jidx = lanes + (16 * k)
            mj_v[s, pl.ds(16 * k, 16)] = jnp.where(
                (jidx >= sts[s]) & (jidx < ens[s]), 1.0, 0.0)

    zeros16 = jnp.zeros((16,), jnp.float32)

    def per_i(ii, _):
        i = wid + _NW * ii

        jlo = jnp.int32(_NPED)
        jhi = jnp.int32(0)
        wi = []
        for s in range(_NSEQ):
            isin = (i >= sts[s]) & (i < ens[s])
            wi.append(jnp.where(isin, 1.0, 0.0))
            jlo = jnp.where(isin, jnp.minimum(jlo, sts[s]), jlo)
            jhi = jnp.where(isin, jnp.maximum(jhi, ens[s]), jhi)
        klo = jlo // 16
        khi = (jhi + 15) // 16

        def zero_body(z, _):
            pacc_v[pl.ds(z * 16, 16)] = zeros16
            return 0

        lax.fori_loop(0, _PACC // 16, zero_body, 0)

        ib = 16 * (i // 16)
        il = i - ib
        pxi = lax.reduce_sum_p.bind(
            jnp.where(lanes == il, px_v[pl.ds(ib, 16)], 0.0), axes=(0,))
        pyi = lax.reduce_sum_p.bind(
            jnp.where(lanes == il, py_v[pl.ds(ib, 16)], 0.0), axes=(0,))

        def per_jvec(k, _):
            j0 = k * 16
            jv = pl.ds(j0, 16)
            jidx = lanes + j0

            wv = zeros16
            for s in range(_NSEQ):
                wv = wv + wi[s] * mj_v[s, jv]

            fx = jnp.clip((px_v[jv] - pxi) / _GSN, -half, half) + half
            fy = jnp.clip((py_v[jv] - pyi) / _GSN, -half, half) + half
            gx = fx.astype(jnp.int32)
            gy = fy.astype(jnp.int32)
            valid = (gx < G) & (gy < G) & (jidx != i)
            cellc = jnp.where(valid, gy * G + gx, G * G + lanes)
            idx0 = cellc * _H

            @plsc.parallel_loop(0, _H, unroll=8)
            def _h_body(h):
                val = hT_v[pl.ds(h * _NPED + j0, 16)] * wv
                plsc.addupdate_scatter(pacc_v, [idx0 + h], val)
            return 0

        lax.fori_loop(klo, khi, per_jvec, 0)

        pltpu.sync_copy(pacc_v.at[pl.ds(0, _G * _G * _H)], pooled_hbm.at[i])
        return 0

    lax.fori_loop(0, _IPW, per_i, 0)


def _linear_kernel(pooled_ref, wt_ref, b_ref, out_ref):
    acc = lax.dot_general(
        pooled_ref[...], wt_ref[...], (((1,), (0,)), ((), ())),
        preferred_element_type=jnp.float32)
    out_ref[...] = jnp.maximum(acc + b_ref[0:1, :], 0.0)


def kernel(hidden_states, seq_start_end, curr_pos, W, b):
    hT = hidden_states.T.reshape(_H * _NPED)
    posT = curr_pos.T
    seq = seq_start_end.astype(jnp.int32).reshape(16)

    mesh = plsc.VectorSubcoreMesh(core_axis_name="c", subcore_axis_name="s")
    pooled = pl.kernel(
        _sc_pool,
        out_type=jax.ShapeDtypeStruct((_NPED, _G * _G * _H), jnp.float32),
        mesh=mesh,
        compiler_params=pltpu.CompilerParams(needs_layout_passes=False),
        scratch_types=[
            pltpu.VMEM((_H * _NPED,), jnp.float32),
            pltpu.VMEM((_NPED,), jnp.float32),
            pltpu.VMEM((_NPED,), jnp.float32),
            pltpu.VMEM((_NSEQ, _NPED), jnp.float32),
            pltpu.VMEM((_PACC,), jnp.float32),
            pltpu.VMEM((16,), jnp.int32),
        ],
    )(hT, posT, seq)

    return pl.pallas_call(
        _linear_kernel,
        out_shape=jax.ShapeDtypeStruct((_NPED, _EMB), jnp.float32),
        in_specs=[
            pl.BlockSpec(memory_space=pltpu.VMEM),
            pl.BlockSpec(memory_space=pltpu.VMEM),
            pl.BlockSpec(memory_space=pltpu.VMEM),
        ],
        out_specs=pl.BlockSpec(memory_space=pltpu.VMEM),
    )(pooled, W.T, b.reshape(1, _EMB))


# speedup vs baseline: 1.9060x; 1.9060x over previous
import functools

import jax
import jax.numpy as jnp
from jax import lax
from jax.experimental import pallas as pl
from jax.experimental.pallas import tpu as pltpu
from jax.experimental.pallas import tpu_sc as plsc

_H = 64
_EMB = 64
_G = 8
_NEIGH = 4.0
_NPED = 512
_NSEQ = 8
_GSN = _NEIGH / (_G - 1)

_NC = 2
_NS = 16
_NW = _NC * _NS
_IPW = _NPED // _NW
_CH = 128
_PROWS = 80


def _sc_pool(hid_hbm, posT_hbm, seq_hbm, pooled_hbm,
             hid_v, px_v, py_v, seq_v, idx_v, zero_v, pacc_sh, sem):
    G = _G
    half = G // 2
    cid = lax.axis_index("c")
    sid = lax.axis_index("s")
    wid = sid * _NC + cid

    pltpu.sync_copy(hid_hbm, hid_v)
    pltpu.sync_copy(posT_hbm.at[0], px_v)
    pltpu.sync_copy(posT_hbm.at[1], py_v)
    pltpu.sync_copy(seq_hbm, seq_v)

    lanes = lax.broadcasted_iota(jnp.int32, (16,), 0)
    seqvec = seq_v[pl.ds(0, 16)]

    def _lane_i32(vec, lane):
        return lax.reduce_sum_p.bind(
            jnp.where(lanes == lane, vec, 0), axes=(0,))

    sts = [_lane_i32(seqvec, 2 * s) for s in range(_NSEQ)]
    ens = [_lane_i32(seqvec, 2 * s + 1) for s in range(_NSEQ)]

    zv = jnp.zeros((16,), jnp.float32)
    for r in range(_PROWS // 4):
        for k in range(_H // 16):
            zero_v[r, pl.ds(16 * k, 16)] = zv

    myacc = pacc_sh.at[sid]

    def per_i(ii, _):
        i = wid + _NW * ii

        for q in range(4):
            pltpu.sync_copy(
                zero_v, myacc.at[pl.ds(q * (_PROWS // 4), _PROWS // 4)])

        ib = 16 * (i // 16)
        il = i - ib
        pxi = lax.reduce_sum_p.bind(
            jnp.where(lanes == il, px_v[pl.ds(ib, 16)], 0.0), axes=(0,))
        pyi = lax.reduce_sum_p.bind(
            jnp.where(lanes == il, py_v[pl.ds(ib, 16)], 0.0), axes=(0,))

        for s in range(_NSEQ):
            st = sts[s]
            en = ens[s]
            isin = (i >= st) & (i < en)

            @pl.when(isin)
            def _seq_pass():
                nch = (en - st + (_CH - 1)) // _CH

                def per_chunk(c, _):
                    jb = jnp.minimum(st + c * _CH, _NPED - _CH)
                    for v in range(_CH // 16):
                        jj = jb + 16 * v + lanes
                        fx = jnp.clip((px_v[pl.ds(jb + 16 * v, 16)] - pxi)
                                      / _GSN, -half, half) + half
                        fy = jnp.clip((py_v[pl.ds(jb + 16 * v, 16)] - pyi)
                                      / _GSN, -half, half) + half
                        gx = fx.astype(jnp.int32)
                        gy = fy.astype(jnp.int32)
                        ok = ((gx < G) & (gy < G) & (jj != i)
                              & (jj >= st) & (jj < en))
                        idx_v[0, pl.ds(16 * v, 16)] = jnp.where(
                            ok, gy * G + gx, G * G + (jj & 15))
                    pltpu.sync_copy(hid_v.at[pl.ds(jb, _CH)],
                                    myacc.at[idx_v.at[0]], add=True)
                    return 0

                lax.fori_loop(0, nch, per_chunk, 0)

        pltpu.sync_copy(myacc.at[pl.ds(0, G * G)], pooled_hbm.at[i])
        return 0

    lax.fori_loop(0, _IPW, per_i, 0)


def _linear_kernel(pooled_ref, wt_ref, b_ref, out_ref):
    acc = lax.dot_general(
        pooled_ref[...], wt_ref[...], (((1,), (0,)), ((), ())),
        preferred_element_type=jnp.float32)
    out_ref[...] = jnp.maximum(acc + b_ref[0:1, :], 0.0)


def kernel(hidden_states, seq_start_end, curr_pos, W, b):
    posT = curr_pos.T
    seq = seq_start_end.astype(jnp.int32).reshape(16)

    mesh = plsc.VectorSubcoreMesh(core_axis_name="c", subcore_axis_name="s")
    pooled = pl.kernel(
        _sc_pool,
        out_type=jax.ShapeDtypeStruct((_NPED, _G * _G, _H), jnp.float32),
        mesh=mesh,
        compiler_params=pltpu.CompilerParams(needs_layout_passes=False),
        scratch_types=[
            pltpu.VMEM((_NPED, _H), jnp.float32),
            pltpu.VMEM((_NPED,), jnp.float32),
            pltpu.VMEM((_NPED,), jnp.float32),
            pltpu.VMEM((16,), jnp.int32),
            pltpu.VMEM((2, _CH), jnp.int32),
            pltpu.VMEM((_PROWS // 4, _H), jnp.float32),
            pltpu.VMEM_SHARED((_NS, _PROWS, _H), jnp.float32),
            pltpu.SemaphoreType.DMA,
        ],
    )(hidden_states, posT, seq)

    return pl.pallas_call(
        _linear_kernel,
        out_shape=jax.ShapeDtypeStruct((_NPED, _EMB), jnp.float32),
        in_specs=[
            pl.BlockSpec(memory_space=pltpu.VMEM),
            pl.BlockSpec(memory_space=pltpu.VMEM),
            pl.BlockSpec(memory_space=pltpu.VMEM),
        ],
        out_specs=pl.BlockSpec(memory_space=pltpu.VMEM),
    )(pooled.reshape(_NPED, _G * _G * _H), W.T, b.reshape(1, _EMB))
